# baseline (speedup 1.0000x reference)
import numpy as np
import jax
import jax.numpy as jnp
from jax.experimental import pallas as pl

_P, _C, _D, _L, _H, _DH, _FF, _NL = 4, 64, 512, 77, 8, 64, 2048, 2
_NPRE, _NCLS, _NSUF = 4, 2, 3
_EOS = 1 + _NPRE + _NCLS + _NSUF
_S = 16
_CB = 16
_B = _CB * _P
_NBLK = _C // _CB
_SCALE = float(1.0 / np.sqrt(_DH))


def _layernorm(h, sc, b):
    m = h.mean(-1, keepdims=True)
    v = ((h - m) ** 2).mean(-1, keepdims=True)
    return (h - m) / jnp.sqrt(v + 1e-5) * sc + b


def _mm(a, b):
    return jax.lax.dot_general(
        a, b, (((a.ndim - 1,), (0,)), ((), ())),
        preferred_element_type=jnp.float32)


def _mmb(a, b):
    return jax.lax.dot_general(
        a.astype(jnp.bfloat16), b, (((a.ndim - 1,), (0,)), ((), ())),
        preferred_element_type=jnp.float32)


def _body(cls_ref, sos_ref, pad_ref, pre_ref, suf_ref, eos_ref, pos_ref,
          ln1s_ref, ln1b_ref, wqkv_ref, bqkv_ref, wo_ref, bo_ref,
          ln2s_ref, ln2b_ref, w1_ref, b1_ref, w2_ref, b2_ref,
          lnfs_ref, lnfb_ref, proj_ref, out_ref):
    cls = cls_ref[...]
    sos = jnp.broadcast_to(sos_ref[...][None], (_B, 1, _D))
    pre = jnp.broadcast_to(pre_ref[...][None], (_CB, _P, _NPRE, _D))
    pre = pre.reshape(_B, _NPRE, _D)
    clsb = jnp.broadcast_to(cls[:, None], (_CB, _P, _NCLS, _D))
    clsb = clsb.reshape(_B, _NCLS, _D)
    suf = jnp.broadcast_to(suf_ref[...][None], (_B, _NSUF, _D))
    eosb = jnp.broadcast_to(eos_ref[...][None], (_CB, _P, _D))
    eosb = eosb.reshape(_B, _D)[:, None, :]
    padb = jnp.broadcast_to(pad_ref[...][None], (_B, _S - _EOS - 1, _D))
    emb = jnp.concatenate([sos, pre, clsb, suf, eosb, padb], axis=1)
    x = (emb + pos_ref[...][None]).reshape(_B * _S, _D)

    r = jax.lax.broadcasted_iota(jnp.int32, (_S, _S), 0)
    c = jax.lax.broadcasted_iota(jnp.int32, (_S, _S), 1)
    neg = jnp.where(c > r, jnp.float32(-1e9), jnp.float32(0.0))

    for l in range(_NL):
        h = _layernorm(x, ln1s_ref[l][None, :], ln1b_ref[l][None, :])
        qkv = _mmb(h, wqkv_ref[l]) + bqkv_ref[l][None, :]
        o_heads = []
        for hh in range(_H):
            q = qkv[:, hh * _DH:(hh + 1) * _DH].reshape(_B, _S, _DH)
            k = qkv[:, _D + hh * _DH:_D + (hh + 1) * _DH].reshape(_B, _S, _DH)
            v = qkv[:, 2 * _D + hh * _DH:2 * _D + (hh + 1) * _DH]
            v = v.reshape(_B, _S, _DH)
            s = jax.lax.dot_general(
                q, k, (((2,), (2,)), ((0,), (0,))),
                preferred_element_type=jnp.float32)
            s = s * _SCALE + neg[None]
            mx = s.max(-1, keepdims=True)
            e = jnp.exp(s - mx)
            att = e / e.sum(-1, keepdims=True)
            oh = jax.lax.dot_general(
                att, v, (((2,), (1,)), ((0,), (0,))),
                preferred_element_type=jnp.float32)
            o_heads.append(oh.reshape(_B * _S, _DH))
        o = jnp.concatenate(o_heads, axis=1)
        x = x + _mmb(o, wo_ref[l]) + bo_ref[l][None, :]
        h2 = _layernorm(x, ln2s_ref[l][None, :], ln2b_ref[l][None, :])
        g = _mmb(h2, w1_ref[l]) + b1_ref[l][None, :]
        g = g * (1.0 / (1.0 + jnp.exp(-1.702 * g)))
        x = x + _mmb(g, w2_ref[l]) + b2_ref[l][None, :]

    x = _layernorm(x, lnfs_ref[...], lnfb_ref[...])
    feats = x.reshape(_B, _S, _D)[:, _EOS, :]
    fm = feats.reshape(_CB, _P, _D).mean(axis=1)
    out_ref[...] = _mmb(fm, proj_ref[...])[None]


def kernel(sos_token, padding_token, prefix_tokens, class_tokens,
           suffix_tokens, eos_tokens, pos_emb, ln1_scale, ln1_bias, Wqkv,
           bqkv, Wo, bo, ln2_scale, ln2_bias, W1, b1, W2, b2, lnf_scale,
           lnf_bias, text_projection):
    pos16 = pos_emb[:_S]
    bf = jnp.bfloat16
    Wqkv = Wqkv.astype(bf)
    Wo = Wo.astype(bf)
    W1 = W1.astype(bf)
    W2 = W2.astype(bf)
    text_projection = text_projection.astype(bf)
    sos2 = sos_token[None]
    pad2 = padding_token[None]
    lnfs2 = lnf_scale[None]
    lnfb2 = lnf_bias[None]

    full = lambda a: pl.BlockSpec(a.shape, lambda i: (0,) * a.ndim)
    operands = (class_tokens, sos2, pad2, prefix_tokens, suffix_tokens,
                eos_tokens, pos16, ln1_scale, ln1_bias, Wqkv, bqkv, Wo, bo,
                ln2_scale, ln2_bias, W1, b1, W2, b2, lnfs2, lnfb2,
                text_projection)
    in_specs = [pl.BlockSpec((_CB, _NCLS, _D), lambda i: (i, 0, 0))]
    in_specs += [full(a) for a in operands[1:]]

    return pl.pallas_call(
        _body,
        grid=(_NBLK,),
        in_specs=in_specs,
        out_specs=pl.BlockSpec((1, _CB, _D), lambda i: (0, i, 0)),
        out_shape=jax.ShapeDtypeStruct((1, _C, _D), jnp.float32),
    )(*operands)


# speedup vs baseline: 1.1555x; 1.1555x over previous
import numpy as np
import jax
import jax.numpy as jnp
from jax.experimental import pallas as pl

_P, _C, _D, _L, _H, _DH, _FF, _NL = 4, 64, 512, 77, 8, 64, 2048, 2
_NPRE, _NCLS, _NSUF = 4, 2, 3
_EOS = 1 + _NPRE + _NCLS + _NSUF
_S = 16
_CB = 16
_B = _CB * _P
_NBLK = _C // _CB
_GR = 128
_GN = _B * _S // _GR
_SCALE = float(1.0 / np.sqrt(_DH))


def _layernorm(h, sc, b):
    m = h.mean(-1, keepdims=True)
    v = ((h - m) ** 2).mean(-1, keepdims=True)
    return (h - m) * jax.lax.rsqrt(v + 1e-5) * sc + b


def _mm(a, b):
    return jax.lax.dot_general(
        a, b, (((a.ndim - 1,), (0,)), ((), ())),
        preferred_element_type=jnp.float32)


def _body(cls_ref, sos_ref, pad_ref, pre_ref, suf_ref, eos_ref, pos_ref,
          ln1s_ref, ln1b_ref, wqkv_ref, bqkv_ref, wo_ref, bo_ref,
          ln2s_ref, ln2b_ref, w1_ref, b1_ref, w2_ref, b2_ref,
          lnfs_ref, lnfb_ref, proj_ref, out_ref):
    cls = cls_ref[...]
    sos = jnp.broadcast_to(sos_ref[...][None], (_B, 1, _D))
    pre = jnp.broadcast_to(pre_ref[...][None], (_CB, _P, _NPRE, _D))
    pre = pre.reshape(_B, _NPRE, _D)
    clsb = jnp.broadcast_to(cls[:, None], (_CB, _P, _NCLS, _D))
    clsb = clsb.reshape(_B, _NCLS, _D)
    suf = jnp.broadcast_to(suf_ref[...][None], (_B, _NSUF, _D))
    eosb = jnp.broadcast_to(eos_ref[...][None], (_CB, _P, _D))
    eosb = eosb.reshape(_B, _D)[:, None, :]
    padb = jnp.broadcast_to(pad_ref[...][None], (_B, _S - _EOS - 1, _D))
    emb = jnp.concatenate([sos, pre, clsb, suf, eosb, padb], axis=1)
    x = (emb + pos_ref[...][None]).reshape(_B * _S, _D)

    r = jax.lax.broadcasted_iota(jnp.int32, (_GR, _GR), 0)
    c = jax.lax.broadcasted_iota(jnp.int32, (_GR, _GR), 1)
    ok = jnp.logical_and(r // _S == c // _S, c <= r)
    neg = jnp.where(ok, jnp.float32(0.0), jnp.float32(-1e9))[None]

    for l in range(_NL):
        h = _layernorm(x, ln1s_ref[l][None, :], ln1b_ref[l][None, :])
        qkv = _mm(h, wqkv_ref[l]) + bqkv_ref[l][None, :]
        o_cols = []
        for hh in range(_H):
            q = qkv[:, hh * _DH:(hh + 1) * _DH]
            k = qkv[:, _D + hh * _DH:_D + (hh + 1) * _DH]
            v = qkv[:, 2 * _D + hh * _DH:2 * _D + (hh + 1) * _DH]
            q = q.reshape(_GN, _GR, _DH)
            k = k.reshape(_GN, _GR, _DH)
            v = v.reshape(_GN, _GR, _DH)
            s = jax.lax.dot_general(
                q, k, (((2,), (2,)), ((0,), (0,))),
                preferred_element_type=jnp.float32)
            s = s * _SCALE + neg
            mx = s.max(-1, keepdims=True)
            e = jnp.exp(s - mx)
            att = e / e.sum(-1, keepdims=True)
            oh = jax.lax.dot_general(
                att, v, (((2,), (1,)), ((0,), (0,))),
                preferred_element_type=jnp.float32)
            o_cols.append(oh.reshape(_B * _S, _DH))
        o = jnp.concatenate(o_cols, axis=1)
        x = x + _mm(o, wo_ref[l]) + bo_ref[l][None, :]
        h2 = _layernorm(x, ln2s_ref[l][None, :], ln2b_ref[l][None, :])
        g = _mm(h2, w1_ref[l]) + b1_ref[l][None, :]
        g = g * (1.0 / (1.0 + jnp.exp(-1.702 * g)))
        x = x + _mm(g, w2_ref[l]) + b2_ref[l][None, :]

    x = _layernorm(x, lnfs_ref[...], lnfb_ref[...])
    feats = x.reshape(_B, _S, _D)[:, _EOS, :]
    fm = feats.reshape(_CB, _P, _D).mean(axis=1)
    out_ref[...] = _mm(fm, proj_ref[...])[None]


def kernel(sos_token, padding_token, prefix_tokens, class_tokens,
           suffix_tokens, eos_tokens, pos_emb, ln1_scale, ln1_bias, Wqkv,
           bqkv, Wo, bo, ln2_scale, ln2_bias, W1, b1, W2, b2, lnf_scale,
           lnf_bias, text_projection):
    pos16 = pos_emb[:_S]
    sos2 = sos_token[None]
    pad2 = padding_token[None]
    lnfs2 = lnf_scale[None]
    lnfb2 = lnf_bias[None]

    full = lambda a: pl.BlockSpec(a.shape, lambda i: (0,) * a.ndim)
    operands = (class_tokens, sos2, pad2, prefix_tokens, suffix_tokens,
                eos_tokens, pos16, ln1_scale, ln1_bias, Wqkv, bqkv, Wo, bo,
                ln2_scale, ln2_bias, W1, b1, W2, b2, lnfs2, lnfb2,
                text_projection)
    in_specs = [pl.BlockSpec((_CB, _NCLS, _D), lambda i: (i, 0, 0))]
    in_specs += [full(a) for a in operands[1:]]

    return pl.pallas_call(
        _body,
        grid=(_NBLK,),
        in_specs=in_specs,
        out_specs=pl.BlockSpec((1, _CB, _D), lambda i: (0, i, 0)),
        out_shape=jax.ShapeDtypeStruct((1, _C, _D), jnp.float32),
    )(*operands)


# speedup vs baseline: 1.3522x; 1.1702x over previous
import numpy as np
import jax
import jax.numpy as jnp
from jax.experimental import pallas as pl

_P, _C, _D, _L, _H, _DH, _FF, _NL = 4, 64, 512, 77, 8, 64, 2048, 2
_NPRE, _NCLS, _NSUF = 4, 2, 3
_EOS = 1 + _NPRE + _NCLS + _NSUF
_S = 12
_CB = 16
_B = _CB * _P
_NBLK = _C // _CB
_GR = 96
_GN = _B * _S // _GR
_SCALE = float(1.0 / np.sqrt(_DH))


def _layernorm(h, sc, b):
    m = h.mean(-1, keepdims=True)
    v = ((h - m) ** 2).mean(-1, keepdims=True)
    return (h - m) * jax.lax.rsqrt(v + 1e-5) * sc + b


def _mm(a, b):
    return jax.lax.dot_general(
        a, b, (((a.ndim - 1,), (0,)), ((), ())),
        preferred_element_type=jnp.float32)


def _body(cls_ref, sos_ref, pad_ref, pre_ref, suf_ref, eos_ref, pos_ref,
          ln1s_ref, ln1b_ref, wqkv_ref, bqkv_ref, wo_ref, bo_ref,
          ln2s_ref, ln2b_ref, w1_ref, b1_ref, w2_ref, b2_ref,
          lnfs_ref, lnfb_ref, proj_ref, out_ref):
    cls = cls_ref[...]
    sos = jnp.broadcast_to(sos_ref[...][None], (_B, 1, _D))
    pre = jnp.broadcast_to(pre_ref[...][None], (_CB, _P, _NPRE, _D))
    pre = pre.reshape(_B, _NPRE, _D)
    clsb = jnp.broadcast_to(cls[:, None], (_CB, _P, _NCLS, _D))
    clsb = clsb.reshape(_B, _NCLS, _D)
    suf = jnp.broadcast_to(suf_ref[...][None], (_B, _NSUF, _D))
    eosb = jnp.broadcast_to(eos_ref[...][None], (_CB, _P, _D))
    eosb = eosb.reshape(_B, _D)[:, None, :]
    padb = jnp.broadcast_to(pad_ref[...][None], (_B, _S - _EOS - 1, _D))
    emb = jnp.concatenate([sos, pre, clsb, suf, eosb, padb], axis=1)
    x = (emb + pos_ref[...][None]).reshape(_B * _S, _D)

    r = jax.lax.broadcasted_iota(jnp.int32, (_GR, _GR), 0)
    c = jax.lax.broadcasted_iota(jnp.int32, (_GR, _GR), 1)
    ok = jnp.logical_and(r // _S == c // _S, c <= r)
    neg = jnp.where(ok, jnp.float32(0.0), jnp.float32(-1e9))[None]

    for l in range(_NL):
        h = _layernorm(x, ln1s_ref[l][None, :], ln1b_ref[l][None, :])
        qkv = _mm(h, wqkv_ref[l]) + bqkv_ref[l][None, :]
        o_cols = []
        for hh in range(_H):
            q = qkv[:, hh * _DH:(hh + 1) * _DH]
            k = qkv[:, _D + hh * _DH:_D + (hh + 1) * _DH]
            v = qkv[:, 2 * _D + hh * _DH:2 * _D + (hh + 1) * _DH]
            q = q.reshape(_GN, _GR, _DH)
            k = k.reshape(_GN, _GR, _DH)
            v = v.reshape(_GN, _GR, _DH)
            s = jax.lax.dot_general(
                q, k, (((2,), (2,)), ((0,), (0,))),
                preferred_element_type=jnp.float32)
            s = s * _SCALE + neg
            mx = s.max(-1, keepdims=True)
            e = jnp.exp(s - mx)
            att = e / e.sum(-1, keepdims=True)
            oh = jax.lax.dot_general(
                att, v, (((2,), (1,)), ((0,), (0,))),
                preferred_element_type=jnp.float32)
            o_cols.append(oh.reshape(_B * _S, _DH))
        o = jnp.concatenate(o_cols, axis=1)
        x = x + _mm(o, wo_ref[l]) + bo_ref[l][None, :]
        h2 = _layernorm(x, ln2s_ref[l][None, :], ln2b_ref[l][None, :])
        g = _mm(h2, w1_ref[l]) + b1_ref[l][None, :]
        g = g * (1.0 / (1.0 + jnp.exp(-1.702 * g)))
        x = x + _mm(g, w2_ref[l]) + b2_ref[l][None, :]

    x = _layernorm(x, lnfs_ref[...], lnfb_ref[...])
    feats = x.reshape(_B, _S, _D)[:, _EOS, :]
    fm = feats.reshape(_CB, _P, _D).mean(axis=1)
    out_ref[...] = _mm(fm, proj_ref[...])[None]


def kernel(sos_token, padding_token, prefix_tokens, class_tokens,
           suffix_tokens, eos_tokens, pos_emb, ln1_scale, ln1_bias, Wqkv,
           bqkv, Wo, bo, ln2_scale, ln2_bias, W1, b1, W2, b2, lnf_scale,
           lnf_bias, text_projection):
    pos16 = pos_emb[:_S]
    sos2 = sos_token[None]
    pad2 = padding_token[None]
    lnfs2 = lnf_scale[None]
    lnfb2 = lnf_bias[None]

    full = lambda a: pl.BlockSpec(a.shape, lambda i: (0,) * a.ndim)
    operands = (class_tokens, sos2, pad2, prefix_tokens, suffix_tokens,
                eos_tokens, pos16, ln1_scale, ln1_bias, Wqkv, bqkv, Wo, bo,
                ln2_scale, ln2_bias, W1, b1, W2, b2, lnfs2, lnfb2,
                text_projection)
    in_specs = [pl.BlockSpec((_CB, _NCLS, _D), lambda i: (i, 0, 0))]
    in_specs += [full(a) for a in operands[1:]]

    return pl.pallas_call(
        _body,
        grid=(_NBLK,),
        in_specs=in_specs,
        out_specs=pl.BlockSpec((1, _CB, _D), lambda i: (0, i, 0)),
        out_shape=jax.ShapeDtypeStruct((1, _C, _D), jnp.float32),
    )(*operands)
